# baseline (speedup 1.0000x reference)
import functools

import jax
import jax.numpy as jnp
from jax.experimental import pallas as pl
from jax.experimental.pallas import tpu as pltpu

N_EXP = 16
TOP_K = 2
D_OUT = 10


def _moe_block_kernel(x_ref, Wg_ref, bg_ref, W1_ref, b1_ref, W2_ref, b2_ref,
                      out_ref):
    x = x_ref[...]
    g = jax.lax.dot_general(
        x, Wg_ref[...], (((1,), (0,)), ((), ()))) + bg_ref[...]
    iota_e = jax.lax.broadcasted_iota(jnp.int32, g.shape, 1)
    v1 = jnp.max(g, axis=1)
    i1 = jnp.min(jnp.where(g == v1[:, None], iota_e, N_EXP), axis=1)
    gm = jnp.where(iota_e == i1[:, None], -jnp.inf, g)
    v2 = jnp.max(gm, axis=1)
    i2 = jnp.min(jnp.where(gm == v2[:, None], iota_e, N_EXP), axis=1)
    e2 = jnp.exp(v2 - v1)
    denom = 1.0 + e2
    w1 = 1.0 / denom
    w2 = e2 / denom

    def body(e, acc):
        h = jax.lax.dot_general(
            x, W1_ref[e], (((1,), (0,)), ((), ()))) + b1_ref[e]
        h = jnp.maximum(h, 0.0)
        o = jax.lax.dot_general(
            h, W2_ref[e], (((1,), (0,)), ((), ()))) + b2_ref[e]
        coef = (jnp.where(i1 == e, w1, 0.0) + jnp.where(i2 == e, w2, 0.0))
        return acc + coef[:, None] * o

    acc = jax.lax.fori_loop(
        0, N_EXP, body, jnp.zeros((x.shape[0], D_OUT), jnp.float32))
    out_ref[...] = acc


@jax.jit
def kernel(x, Wg, bg, W1, b1, W2, b2):
    n_tok, d_in = x.shape
    bn = min(1024, n_tok)
    grid = (n_tok // bn,)
    return pl.pallas_call(
        _moe_block_kernel,
        grid=grid,
        in_specs=[
            pl.BlockSpec((bn, d_in), lambda i: (i, 0)),
            pl.BlockSpec(Wg.shape, lambda i: (0, 0)),
            pl.BlockSpec((1, N_EXP), lambda i: (0, 0)),
            pl.BlockSpec(W1.shape, lambda i: (0, 0, 0)),
            pl.BlockSpec((N_EXP, 1, b1.shape[-1]), lambda i: (0, 0, 0)),
            pl.BlockSpec(W2.shape, lambda i: (0, 0, 0)),
            pl.BlockSpec((N_EXP, 1, D_OUT), lambda i: (0, 0, 0)),
        ],
        out_specs=pl.BlockSpec((bn, D_OUT), lambda i: (i, 0)),
        out_shape=jax.ShapeDtypeStruct((n_tok, D_OUT), jnp.float32),
        compiler_params=pltpu.CompilerParams(
            dimension_semantics=("arbitrary",)),
    )(x, Wg, bg.reshape(1, N_EXP), W1, b1.reshape(N_EXP, 1, -1), W2,
      b2.reshape(N_EXP, 1, D_OUT))


# speedup vs baseline: 2.3553x; 2.3553x over previous
import jax
import jax.numpy as jnp
from jax.experimental import pallas as pl
from jax.experimental.pallas import tpu as pltpu

N_EXP = 16
D_HID = 64
D_OUT = 10


def _dot(a, b):
    return jax.lax.dot_general(a, b, (((1,), (0,)), ((), ())))


def _moe_block_kernel(x_ref, Wg_ref, bg_ref, W1_ref, b1_ref, W2_ref, b2_ref,
                      out_ref):
    x = x_ref[...]
    g = _dot(x, Wg_ref[...]) + bg_ref[...]
    iota_e = jax.lax.broadcasted_iota(jnp.int32, g.shape, 1)
    v1 = jnp.max(g, axis=1)
    i1 = jnp.min(jnp.where(g == v1[:, None], iota_e, N_EXP), axis=1)
    gm = jnp.where(iota_e == i1[:, None], -jnp.inf, g)
    v2 = jnp.max(gm, axis=1)
    i2 = jnp.min(jnp.where(gm == v2[:, None], iota_e, N_EXP), axis=1)
    e2 = jnp.exp(v2 - v1)
    denom = 1.0 + e2
    coef = (jnp.where(iota_e == i1[:, None], 1.0, 0.0)
            + jnp.where(iota_e == i2[:, None], e2[:, None], 0.0)) / denom[:, None]

    h = jnp.maximum(_dot(x, W1_ref[...]) + b1_ref[...], 0.0)
    r16 = jax.lax.broadcasted_iota(jnp.int32, (N_EXP, N_EXP * D_HID), 0)
    c1024 = jax.lax.broadcasted_iota(jnp.int32, (N_EXP, N_EXP * D_HID), 1)
    expand = jnp.where(c1024 // D_HID == r16, 1.0, 0.0)
    hs = h * _dot(coef, expand)
    out_ref[...] = _dot(hs, W2_ref[...]) + _dot(coef, b2_ref[...])


@jax.jit
def kernel(x, Wg, bg, W1, b1, W2, b2):
    n_tok, d_in = x.shape
    eh = N_EXP * D_HID
    W1all = W1.transpose(1, 0, 2).reshape(d_in, eh)
    b1all = b1.reshape(1, eh)
    W2stack = W2.reshape(eh, D_OUT)
    bn = min(1024, n_tok)
    grid = (n_tok // bn,)
    return pl.pallas_call(
        _moe_block_kernel,
        grid=grid,
        in_specs=[
            pl.BlockSpec((bn, d_in), lambda i: (i, 0)),
            pl.BlockSpec(Wg.shape, lambda i: (0, 0)),
            pl.BlockSpec((1, N_EXP), lambda i: (0, 0)),
            pl.BlockSpec((d_in, eh), lambda i: (0, 0)),
            pl.BlockSpec((1, eh), lambda i: (0, 0)),
            pl.BlockSpec((eh, D_OUT), lambda i: (0, 0)),
            pl.BlockSpec((N_EXP, D_OUT), lambda i: (0, 0)),
        ],
        out_specs=pl.BlockSpec((bn, D_OUT), lambda i: (i, 0)),
        out_shape=jax.ShapeDtypeStruct((n_tok, D_OUT), jnp.float32),
        compiler_params=pltpu.CompilerParams(
            dimension_semantics=("arbitrary",)),
    )(x, Wg, bg.reshape(1, N_EXP), W1all, b1all, W2stack, b2)


# speedup vs baseline: 4.2929x; 1.8226x over previous
import jax
import jax.numpy as jnp
from jax.experimental import pallas as pl
from jax.experimental.pallas import tpu as pltpu

N_EXP = 16
D_HID = 64
D_OUT = 10


def _moe_block_kernel(xT_ref, Wg_ref, bg_ref, W1_ref, b1_ref, W2_ref, b2_ref,
                      ex_ref, outT_ref):
    xT = xT_ref[...]
    g = jax.lax.dot_general(
        xT, Wg_ref[...], (((0,), (0,)), ((), ()))) + bg_ref[...]
    v1 = jnp.max(g, axis=1, keepdims=True)
    top_cnt = jnp.sum(jnp.where(g == v1, 1.0, 0.0), axis=1, keepdims=True)
    v2 = jnp.max(jnp.where(g < v1, g, -jnp.inf), axis=1, keepdims=True)
    v2 = jnp.where(top_cnt > 1.0, v1, v2)
    z = jnp.where(g >= v2, jnp.exp(g - v1), 0.0)
    coef = z / jnp.sum(z, axis=1, keepdims=True)

    xb = xT.astype(jnp.bfloat16)
    h = jnp.maximum(
        jax.lax.dot_general(xb, W1_ref[...], (((0,), (0,)), ((), ())),
                            preferred_element_type=jnp.float32)
        + b1_ref[...], 0.0)
    ce = jax.lax.dot_general(coef, ex_ref[...], (((1,), (0,)), ((), ())))
    hs = (h * ce).astype(jnp.bfloat16)
    outT = (jax.lax.dot_general(W2_ref[...], hs, (((0,), (1,)), ((), ())),
                                preferred_element_type=jnp.float32)
            + jax.lax.dot_general(b2_ref[...], coef, (((0,), (1,)), ((), ()))))
    outT_ref[...] = outT


@jax.jit
def kernel(x, Wg, bg, W1, b1, W2, b2):
    n_tok, d_in = x.shape
    eh = N_EXP * D_HID
    xT = x.T
    W1all = W1.transpose(1, 0, 2).reshape(d_in, eh).astype(jnp.bfloat16)
    b1all = b1.reshape(1, eh)
    W2stack = W2.reshape(eh, D_OUT).astype(jnp.bfloat16)
    expand = jnp.repeat(jnp.eye(N_EXP, dtype=jnp.float32), D_HID, axis=1)
    bn = min(2048, n_tok)
    grid = (n_tok // bn,)
    outT = pl.pallas_call(
        _moe_block_kernel,
        grid=grid,
        in_specs=[
            pl.BlockSpec((d_in, bn), lambda i: (0, i)),
            pl.BlockSpec(Wg.shape, lambda i: (0, 0)),
            pl.BlockSpec((1, N_EXP), lambda i: (0, 0)),
            pl.BlockSpec((d_in, eh), lambda i: (0, 0)),
            pl.BlockSpec((1, eh), lambda i: (0, 0)),
            pl.BlockSpec((eh, D_OUT), lambda i: (0, 0)),
            pl.BlockSpec((N_EXP, D_OUT), lambda i: (0, 0)),
            pl.BlockSpec((N_EXP, eh), lambda i: (0, 0)),
        ],
        out_specs=pl.BlockSpec((D_OUT, bn), lambda i: (0, i)),
        out_shape=jax.ShapeDtypeStruct((D_OUT, n_tok), jnp.float32),
        compiler_params=pltpu.CompilerParams(
            dimension_semantics=("arbitrary",)),
    )(xT, Wg, bg.reshape(1, N_EXP), W1all, b1all, W2stack, b2, expand)
    return outT.T
